# baseline (speedup 1.0000x reference)
import functools

import jax
import jax.numpy as jnp
from jax import lax
from jax.experimental import pallas as pl
from jax.experimental.pallas import tpu as pltpu
from jax.experimental.pallas import tpu_sc as plsc

TOP_K = 8
BLOCK_M = 1024


def _matmul_body(x_ref, w_ref, h_ref):
    h_ref[...] = jnp.dot(x_ref[...], w_ref[...],
                         preferred_element_type=jnp.float32)


def _gate_logits(xf, W_gate):
    t, d_model = xf.shape
    n_experts = W_gate.shape[-1]
    bm = min(BLOCK_M, t)
    return pl.pallas_call(
        _matmul_body,
        grid=(t // bm,),
        in_specs=[
            pl.BlockSpec((bm, d_model), lambda i: (i, 0)),
            pl.BlockSpec((d_model, n_experts), lambda i: (0, 0)),
        ],
        out_specs=pl.BlockSpec((bm, n_experts), lambda i: (i, 0)),
        out_shape=jax.ShapeDtypeStruct((t, n_experts), jnp.float32),
    )(xf, W_gate)


def _gather16(v, idx):
    dnums = lax.GatherDimensionNumbers(
        offset_dims=(), collapsed_slice_dims=(0,), start_index_map=(0,))
    return lax.gather(v, idx[:, None], dnums, (1,),
                      mode=lax.GatherScatterMode.PROMISE_IN_BOUNDS)


def _merge_top16(a, b, iota):
    h = jnp.maximum(a, lax.rev(b, (0,)))
    for stride in (8, 4, 2, 1):
        p = _gather16(h, iota ^ stride)
        keep_max = (iota & stride) == 0
        h = jnp.where(keep_max, jnp.maximum(h, p), jnp.minimum(h, p))
    return h


def _sc_router_body(h_hbm, dw_hbm, idx_hbm, parts_hbm,
                    h_v, dw_v, idx_v, util_v, stage_v):
    nc = 2
    wid = lax.axis_index("s") * nc + lax.axis_index("c")
    tpw = h_v.shape[0]
    base = wid * tpw
    iota = lax.iota(jnp.int32, 16)
    min_mask = jnp.int32(~63)

    pltpu.sync_copy(h_hbm.at[pl.ds(base, tpw)], h_v)

    zeros16 = jnp.zeros((16,), jnp.float32)
    for j in range(4):
        util_v[pl.ds(16 * j, 16)] = zeros16

    def body(tok, imp):
        lg = [h_v[tok, pl.ds(16 * j, 16)] for j in range(4)]
        ks = []
        for j in range(4):
            b = lax.bitcast_convert_type(lg[j], jnp.int32)
            kb = jnp.where(b < 0, b ^ jnp.int32(0x7FFFFFFF), b)
            ks.append((kb & min_mask) | (63 - (iota + 16 * j)))
        ss = [plsc.sort_key_val(k, k, descending=True)[0] for k in ks]
        t01 = _merge_top16(ss[0], ss[1], iota)
        t23 = _merge_top16(ss[2], ss[3], iota)
        top = _merge_top16(t01, t23, iota)
        idx8 = 63 - (top & 63)
        tb = top & min_mask
        vb = jnp.where(tb < 0, tb ^ jnp.int32(0x7FFFFFFF), tb)
        vals = lax.bitcast_convert_type(vb, jnp.float32)
        v0 = jnp.max(vals)
        e = jnp.where(iota < TOP_K, jnp.exp(vals - v0), 0.0)
        dw = e / jnp.sum(e)
        mask8 = iota < TOP_K
        plsc.store_compressed(dw_v.at[pl.ds(tok * TOP_K, 16)], dw, mask=mask8)
        plsc.store_compressed(idx_v.at[pl.ds(tok * TOP_K, 16)], idx8, mask=mask8)
        plsc.addupdate_scatter(util_v, [idx8], jnp.ones((16,), jnp.float32),
                               mask=mask8)
        p = [jnp.exp(g - v0) for g in lg]
        s64 = jnp.sum(p[0]) + jnp.sum(p[1]) + jnp.sum(p[2]) + jnp.sum(p[3])
        return tuple(imp[j] + p[j] / s64 for j in range(4))

    imp = lax.fori_loop(0, tpw, body, (zeros16,) * 4)

    pltpu.sync_copy(dw_v.at[pl.ds(0, tpw * TOP_K)],
                    dw_hbm.at[pl.ds(base * TOP_K, tpw * TOP_K)])
    pltpu.sync_copy(idx_v.at[pl.ds(0, tpw * TOP_K)],
                    idx_hbm.at[pl.ds(base * TOP_K, tpw * TOP_K)])

    for j in range(4):
        stage_v[pl.ds(16 * j, 16)] = util_v[pl.ds(16 * j, 16)]
        stage_v[pl.ds(64 + 16 * j, 16)] = imp[j]
    pltpu.sync_copy(stage_v, parts_hbm.at[wid])


def _sc_route(h):
    t = h.shape[0]
    nw = 32
    tpw = t // nw
    mesh = plsc.VectorSubcoreMesh(core_axis_name="c", subcore_axis_name="s")
    f = functools.partial(
        pl.kernel, mesh=mesh,
        compiler_params=pltpu.CompilerParams(needs_layout_passes=False),
        out_type=[
            jax.ShapeDtypeStruct((t * TOP_K,), jnp.float32),
            jax.ShapeDtypeStruct((t * TOP_K,), jnp.int32),
            jax.ShapeDtypeStruct((nw, 128), jnp.float32),
        ],
        scratch_types=[
            pltpu.VMEM((tpw, 64), jnp.float32),
            pltpu.VMEM((tpw * TOP_K + 8,), jnp.float32),
            pltpu.VMEM((tpw * TOP_K + 8,), jnp.int32),
            pltpu.VMEM((64,), jnp.float32),
            pltpu.VMEM((128,), jnp.float32),
        ],
    )(_sc_router_body)
    return f(h)


def _aux_body(parts_ref, aux_ref):
    parts = parts_ref[...]
    util = jnp.sum(parts[:, :64], axis=0)
    imp = jnp.sum(parts[:, 64:], axis=0)

    def cv(v):
        mean = jnp.sum(v) / 64.0
        var = jnp.sum((v - mean) ** 2) / 63.0
        return jnp.sqrt(var) / (mean + 1e-6)

    val = (cv(util) + cv(imp)) * 0.01
    aux_ref[...] = jnp.full((1, 1), val, jnp.float32)


def _aux_loss(parts):
    return pl.pallas_call(
        _aux_body,
        out_shape=jax.ShapeDtypeStruct((1, 1), jnp.float32),
    )(parts)


def kernel(x, W_gate, W_noise):
    orig_shape = x.shape
    d_model = x.shape[-1]
    xf = x.reshape(-1, d_model)
    t = xf.shape[0]
    n_chunks = 2
    tc = t // n_chunks
    hs = [_gate_logits(lax.slice_in_dim(xf, i * tc, (i + 1) * tc), W_gate)
          for i in range(n_chunks)]
    routed = [_sc_route(h) for h in hs]
    dw = jnp.concatenate([r[0] for r in routed])
    idxs = jnp.concatenate([r[1] for r in routed])
    parts = jnp.concatenate([r[2] for r in routed])
    aux = _aux_loss(parts)
    return (dw.reshape(orig_shape[:-1] + (TOP_K,)),
            idxs.reshape(orig_shape[:-1] + (TOP_K,)),
            aux[0, 0])


# speedup vs baseline: 3.5353x; 3.5353x over previous
import functools

import jax
import jax.numpy as jnp
from jax.experimental import pallas as pl
from jax.experimental.pallas import tpu as pltpu

TOP_K = 8
BLOCK_M = 1024


def _router_body(x_ref, w_ref, dw_ref, idx_ref, aux_ref, util_acc, imp_acc,
                 *, n_experts):
    i = pl.program_id(0)
    n = pl.num_programs(0)

    h = jnp.dot(x_ref[...], w_ref[...], preferred_element_type=jnp.float32)
    ht = h.T

    bits = jax.lax.bitcast_convert_type(ht, jnp.int32)
    key = jnp.where(bits < 0, bits ^ jnp.int32(0x7FFFFFFF), bits)
    expert = jax.lax.broadcasted_iota(jnp.int32, ht.shape, 0)
    kk = (key & jnp.int32(~63)) | (63 - expert)

    min32 = jnp.int32(-2**31)
    tops = []
    for _ in range(TOP_K):
        m = jnp.max(kk, axis=0, keepdims=True)
        kk = jnp.where(kk == m, min32, kk)
        tops.append(m)

    keys8 = jnp.concatenate(tops, axis=0)
    idxs = 63 - (keys8 & 63)
    tb = keys8 & jnp.int32(~63)
    vb = jnp.where(tb < 0, tb ^ jnp.int32(0x7FFFFFFF), tb)
    vals = jax.lax.bitcast_convert_type(vb, jnp.float32)

    e = jnp.exp(vals - vals[0:1, :])
    dw_ref[...] = e / jnp.sum(e, axis=0, keepdims=True)
    idx_ref[...] = idxs

    p = jnp.exp(ht - vals[0:1, :])
    p = p / jnp.sum(p, axis=0, keepdims=True)
    imp_part = jnp.sum(p, axis=1, keepdims=True)
    util_part = jnp.sum(jnp.where(kk == min32, 1.0, 0.0),
                        axis=1, keepdims=True)

    @pl.when(i == 0)
    def _():
        util_acc[...] = jnp.zeros_like(util_acc)
        imp_acc[...] = jnp.zeros_like(imp_acc)

    util_acc[...] += util_part
    imp_acc[...] += imp_part

    @pl.when(i == n - 1)
    def _():
        def cv(v):
            mean = jnp.sum(v) / n_experts
            var = jnp.sum((v - mean) ** 2) / (n_experts - 1)
            return jnp.sqrt(var) / (mean + 1e-6)
        val = (cv(util_acc[...]) + cv(imp_acc[...])) * 0.01
        aux_ref[...] = jnp.full((1, 1), val, jnp.float32)


def kernel(x, W_gate, W_noise):
    orig_shape = x.shape
    d_model = x.shape[-1]
    n_experts = W_gate.shape[-1]
    xf = x.reshape(-1, d_model)
    t = xf.shape[0]
    bm = min(BLOCK_M, t)
    grid = t // bm

    dw_t, idx_t, aux = pl.pallas_call(
        functools.partial(_router_body, n_experts=n_experts),
        grid=(grid,),
        in_specs=[
            pl.BlockSpec((bm, d_model), lambda i: (i, 0)),
            pl.BlockSpec((d_model, n_experts), lambda i: (0, 0)),
        ],
        out_specs=[
            pl.BlockSpec((TOP_K, bm), lambda i: (0, i)),
            pl.BlockSpec((TOP_K, bm), lambda i: (0, i)),
            pl.BlockSpec((1, 1), lambda i: (0, 0)),
        ],
        out_shape=[
            jax.ShapeDtypeStruct((TOP_K, t), jnp.float32),
            jax.ShapeDtypeStruct((TOP_K, t), jnp.int32),
            jax.ShapeDtypeStruct((1, 1), jnp.float32),
        ],
        scratch_shapes=[
            pltpu.VMEM((n_experts, 1), jnp.float32),
            pltpu.VMEM((n_experts, 1), jnp.float32),
        ],
    )(xf, W_gate)

    return (dw_t.T.reshape(orig_shape[:-1] + (TOP_K,)),
            idx_t.T.reshape(orig_shape[:-1] + (TOP_K,)),
            aux[0, 0])
